# Initial kernel scaffold; baseline (speedup 1.0000x reference)
import jax
import jax.numpy as jnp
from jax.experimental import pallas as pl


def kernel(in_snd_slice, quant_onehot):
    raise NotImplementedError("write your pallas kernel here")


# speedup vs baseline: 7.9273x; 7.9273x over previous
import jax
import jax.numpy as jnp
from jax.experimental import pallas as pl

_NQ = 256


def _body(idx_ref, out_ref):
    tb = out_ref.shape[2]
    iota = jax.lax.broadcasted_iota(jnp.int32, (_NQ, tb), 0)
    out_ref[0] = (idx_ref[0] == iota).astype(jnp.float32)


def kernel(in_snd_slice, quant_onehot):
    idx = in_snd_slice.astype(jnp.int32)
    B, T = idx.shape
    TB = 1024
    idx3 = idx.reshape(B, 1, T)
    out = pl.pallas_call(
        _body,
        grid=(B, T // TB),
        in_specs=[pl.BlockSpec((1, 1, TB), lambda b, t: (b, 0, t))],
        out_specs=pl.BlockSpec((1, _NQ, TB), lambda b, t: (b, 0, t)),
        out_shape=jax.ShapeDtypeStruct((B, _NQ, T), jnp.float32),
    )(idx3)
    return out
